# baseline (speedup 1.0000x reference)
import functools

import jax
import jax.numpy as jnp
import numpy as np
from jax import lax
from jax.experimental import pallas as pl
from jax.experimental.pallas import tpu as pltpu
from jax.experimental.pallas import tpu_sc as plsc

N_CLASSES = 90
N_LEVELS = 5
IMG = 512
MIN_LVL = 3
A_SCALE = 4.0
N_SCALES = 3
ASP = [(1.0, 1.0), (1.4, 0.7), (0.7, 1.4)]
N_TOP = 5000
N_DET = 100
IOU_T = 0.5
NB = 8
NP_PAD = 5120
N_ANCH = 49104

_NC = 2
_NS = 16
_NW = _NC * _NS
G_TOT = NB * N_TOP
G_PAD = 40960
G_PER_W = G_PAD // _NW
G_CHUNK = 128
G_NCH = G_PER_W // G_CHUNK


def _anchor_table():
    all_boxes = []
    for level in range(MIN_LVL, MIN_LVL + N_LEVELS):
        stride = 2 ** level
        feat = IMG // stride
        yc = (np.arange(feat) + 0.5) * stride
        xc = (np.arange(feat) + 0.5) * stride
        yv, xv = np.meshgrid(yc, xc, indexing='ij')
        yv = yv.reshape(-1)
        xv = xv.reshape(-1)
        per_cfg = []
        for octave in range(N_SCALES):
            scale = 2.0 ** (octave / float(N_SCALES))
            base = A_SCALE * stride * scale
            for ay, ax in ASP:
                h2 = base * ay / 2.0
                w2 = base * ax / 2.0
                per_cfg.append(
                    np.stack([yv - h2, xv - w2, yv + h2, xv + w2], axis=1)[:, None, :])
        all_boxes.append(np.concatenate(per_cfg, axis=1).reshape(-1, 4))
    return np.concatenate(all_boxes, axis=0).astype(np.float32)

_ANCH = _anchor_table()


def _sc_gather_body(b0, b1, b2, b3, a0, a1, a2, a3, bidx, aidx,
                    o0, o1, o2, o3, o4, o5, o6, o7,
                    bidx_v, aidx_v, v0, v1, v2, v3, v4, v5, v6, v7,
                    bsem, asem):
    wid = lax.axis_index("s") * _NC + lax.axis_index("c")
    base = wid * G_PER_W
    pltpu.sync_copy(bidx.at[wid], bidx_v)
    pltpu.sync_copy(aidx.at[wid], aidx_v)
    btabs = [b0, b1, b2, b3]
    atabs = [a0, a1, a2, a3]
    bvals = [v0, v1, v2, v3]
    avals = [v4, v5, v6, v7]

    def chunk(j, _):
        sl = pl.ds(j * G_CHUNK, G_CHUNK)
        waits = []
        for t in range(4):
            waits.append(
                pltpu.async_copy(btabs[t].at[bidx_v.at[j]], bvals[t].at[sl], bsem))
            waits.append(
                pltpu.async_copy(atabs[t].at[aidx_v.at[j]], avals[t].at[sl], asem))
        for wct in waits:
            wct.wait()
        return 0

    lax.fori_loop(0, G_NCH, chunk, 0)
    outs = [o0, o1, o2, o3, o4, o5, o6, o7]
    for t in range(8):
        pltpu.sync_copy((bvals + avals)[t], outs[t].at[pl.ds(base, G_PER_W)])


_SC_OUT = tuple(jax.ShapeDtypeStruct((G_PAD,), jnp.float32) for _ in range(8))


@functools.lru_cache(maxsize=1)
def _sc_gather_fn():
    return functools.partial(
        pl.kernel,
        mesh=plsc.VectorSubcoreMesh(core_axis_name="c", subcore_axis_name="s"),
        out_type=_SC_OUT,
        scratch_types=[
            pltpu.VMEM((G_NCH, G_CHUNK), jnp.int32),
            pltpu.VMEM((G_NCH, G_CHUNK), jnp.int32),
        ] + [pltpu.VMEM((G_PER_W,), jnp.float32)] * 8 + [
            pltpu.SemaphoreType.DMA,
            pltpu.SemaphoreType.DMA,
        ],
    )(_sc_gather_body)


def _nms_body(ty_ref, tx_ref, th_ref, tw_ref,
              ay1_ref, ax1_ref, ay2_ref, ax2_ref,
              logit_ref, cls_ref, scale_ref, out_ref):
    ty = ty_ref[...]
    tx = tx_ref[...]
    th = th_ref[...]
    tw = tw_ref[...]
    ay1 = ay1_ref[...]
    ax1 = ax1_ref[...]
    ay2 = ay2_ref[...]
    ax2 = ax2_ref[...]
    clsf = cls_ref[...]

    yca = (ay1 + ay2) * 0.5
    xca = (ax1 + ax2) * 0.5
    ha = ay2 - ay1
    wa = ax2 - ax1
    h = jnp.exp(th) * ha
    w = jnp.exp(tw) * wa
    yc = ty * ha + yca
    xc = tx * wa + xca
    by1 = yc - h * 0.5
    bx1 = xc - w * 0.5
    by2 = yc + h * 0.5
    bx2 = xc + w * 0.5

    lane = jax.lax.broadcasted_iota(jnp.int32, (NB, NP_PAD), 1)
    valid = lane < N_TOP
    scores0 = jnp.where(valid, jax.nn.sigmoid(logit_ref[...]), -1e30)

    off = clsf * 1e4
    oy1 = by1 + off
    ox1 = bx1 + off
    oy2 = by2 + off
    ox2 = bx2 + off
    area_o = (oy2 - oy1) * (ox2 - ox1)
    scale = scale_ref[...]

    def step(i, scores):
        m = jnp.max(scores, axis=1, keepdims=True)
        cand = jnp.where(scores == m, lane, NP_PAD)
        bidx = jnp.min(cand, axis=1, keepdims=True)
        sel = lane == bidx

        def pick(a):
            return jnp.sum(jnp.where(sel, a, 0.0), axis=1, keepdims=True)

        b_oy1 = pick(oy1)
        b_ox1 = pick(ox1)
        b_oy2 = pick(oy2)
        b_ox2 = pick(ox2)
        ya = jnp.maximum(b_oy1, oy1)
        xa = jnp.maximum(b_ox1, ox1)
        yb = jnp.minimum(b_oy2, oy2)
        xb = jnp.minimum(b_ox2, ox2)
        inter = jnp.maximum(yb - ya, 0.0) * jnp.maximum(xb - xa, 0.0)
        a1 = (b_oy2 - b_oy1) * (b_ox2 - b_ox1)
        iou = inter / (a1 + area_o - inter + 1e-8)

        b_score = pick(scores)
        det = jnp.concatenate([
            pick(by1) * scale, pick(bx1) * scale,
            pick(by2) * scale, pick(bx2) * scale,
            b_score, pick(clsf) + 1.0,
        ], axis=1)
        out_ref[:, pl.ds(i, 1), :] = det[:, None, :]

        scores = jnp.where(iou > IOU_T, -1.0, scores)
        return jnp.where(sel, -1.0, scores)

    jax.lax.fori_loop(0, N_DET, step, scores0)


def _pad_np(x):
    return jnp.pad(x, ((0, 0), (0, NP_PAD - N_TOP)))


@jax.jit
def kernel(cls_p3, cls_p4, cls_p5, cls_p6, cls_p7,
           box_p3, box_p4, box_p5, box_p6, box_p7, image_scales):
    cls_list = [cls_p3, cls_p4, cls_p5, cls_p6, cls_p7]
    box_list = [box_p3, box_p4, box_p5, box_p6, box_p7]
    b = cls_p3.shape[0]
    cls_flat = jnp.concatenate(
        [jnp.transpose(c, (0, 2, 3, 1)).reshape(b, -1) for c in cls_list], axis=1)
    top_v, topi = jax.lax.top_k(cls_flat, N_TOP)
    indices = topi // N_CLASSES
    classes = topi % N_CLASSES

    pad_fill = jnp.arange(G_PAD - G_TOT, dtype=jnp.int32)
    aidx = jnp.concatenate([indices.reshape(-1), pad_fill])
    bidx = jnp.concatenate(
        [(indices + jnp.arange(b, dtype=jnp.int32)[:, None] * N_ANCH).reshape(-1),
         pad_fill])
    btabs = []
    for j in range(4):
        btabs.append(jnp.concatenate(
            [jnp.transpose(o[:, j::4], (0, 2, 3, 1)).reshape(b, -1)
             for o in box_list], axis=1).reshape(-1))
    atabs = [jnp.asarray(np.ascontiguousarray(_ANCH[:, j])) for j in range(4)]
    gat = _sc_gather_fn()(*btabs, *atabs,
                     bidx.reshape(_NW, G_NCH, G_CHUNK),
                     aidx.reshape(_NW, G_NCH, G_CHUNK))

    def unflat(x):
        return x[:G_TOT].reshape(b, N_TOP)

    args = [
        unflat(gat[0]), unflat(gat[1]), unflat(gat[2]), unflat(gat[3]),
        unflat(gat[4]), unflat(gat[5]), unflat(gat[6]), unflat(gat[7]),
        top_v, classes.astype(jnp.float32),
        image_scales[:, None],
    ]
    args = [_pad_np(a) if a.shape == (NB, N_TOP) else a for a in args]
    dets = pl.pallas_call(
        _nms_body,
        out_shape=jax.ShapeDtypeStruct((NB, N_DET, 6), jnp.float32),
    )(*args)
    return dets


# speedup vs baseline: 1.8939x; 1.8939x over previous
import functools

import jax
import jax.numpy as jnp
import numpy as np
from jax import lax
from jax.experimental import pallas as pl
from jax.experimental.pallas import tpu as pltpu
from jax.experimental.pallas import tpu_sc as plsc

N_CLASSES = 90
N_LEVELS = 5
IMG = 512
MIN_LVL = 3
A_SCALE = 4.0
N_SCALES = 3
ASP = [(1.0, 1.0), (1.4, 0.7), (0.7, 1.4)]
N_TOP = 5000
N_DET = 100
IOU_T = 0.5
NB = 8
NP_PAD = 5000
N_ANCH = 49104

_NC = 2
_NS = 16
_NW = _NC * _NS
G_TOT = NB * N_TOP
G_PAD = 40960
G_PER_W = G_PAD // _NW
G_CHUNK = 128
G_NCH = G_PER_W // G_CHUNK


def _anchor_table():
    all_boxes = []
    for level in range(MIN_LVL, MIN_LVL + N_LEVELS):
        stride = 2 ** level
        feat = IMG // stride
        yc = (np.arange(feat) + 0.5) * stride
        xc = (np.arange(feat) + 0.5) * stride
        yv, xv = np.meshgrid(yc, xc, indexing='ij')
        yv = yv.reshape(-1)
        xv = xv.reshape(-1)
        per_cfg = []
        for octave in range(N_SCALES):
            scale = 2.0 ** (octave / float(N_SCALES))
            base = A_SCALE * stride * scale
            for ay, ax in ASP:
                h2 = base * ay / 2.0
                w2 = base * ax / 2.0
                per_cfg.append(
                    np.stack([yv - h2, xv - w2, yv + h2, xv + w2], axis=1)[:, None, :])
        all_boxes.append(np.concatenate(per_cfg, axis=1).reshape(-1, 4))
    return np.concatenate(all_boxes, axis=0).astype(np.float32)

_ANCH = _anchor_table()


def _sc_gather_body(b0, b1, b2, b3, a0, a1, a2, a3, bidx, aidx,
                    o0, o1, o2, o3, o4, o5, o6, o7,
                    bidx_v, aidx_v, v0, v1, v2, v3, v4, v5, v6, v7,
                    bsem, asem):
    wid = lax.axis_index("s") * _NC + lax.axis_index("c")
    base = wid * G_PER_W
    pltpu.sync_copy(bidx.at[wid], bidx_v)
    pltpu.sync_copy(aidx.at[wid], aidx_v)
    btabs = [b0, b1, b2, b3]
    atabs = [a0, a1, a2, a3]
    bvals = [v0, v1, v2, v3]
    avals = [v4, v5, v6, v7]

    def chunk(j, _):
        sl = pl.ds(j * G_CHUNK, G_CHUNK)
        waits = []
        for t in range(4):
            waits.append(
                pltpu.async_copy(btabs[t].at[bidx_v.at[j]], bvals[t].at[sl], bsem))
            waits.append(
                pltpu.async_copy(atabs[t].at[aidx_v.at[j]], avals[t].at[sl], asem))
        for wct in waits:
            wct.wait()
        return 0

    lax.fori_loop(0, G_NCH, chunk, 0)
    outs = [o0, o1, o2, o3, o4, o5, o6, o7]
    for t in range(8):
        pltpu.sync_copy((bvals + avals)[t], outs[t].at[pl.ds(base, G_PER_W)])


_SC_OUT = tuple(jax.ShapeDtypeStruct((G_PAD,), jnp.float32) for _ in range(8))


@functools.lru_cache(maxsize=1)
def _sc_gather_fn():
    return functools.partial(
        pl.kernel,
        mesh=plsc.VectorSubcoreMesh(core_axis_name="c", subcore_axis_name="s"),
        out_type=_SC_OUT,
        scratch_types=[
            pltpu.VMEM((G_NCH, G_CHUNK), jnp.int32),
            pltpu.VMEM((G_NCH, G_CHUNK), jnp.int32),
        ] + [pltpu.VMEM((G_PER_W,), jnp.float32)] * 8 + [
            pltpu.SemaphoreType.DMA,
            pltpu.SemaphoreType.DMA,
        ],
    )(_sc_gather_body)


def _nms_body(ty_ref, tx_ref, th_ref, tw_ref,
              ay1_ref, ax1_ref, ay2_ref, ax2_ref,
              logit_ref, cls_ref, scale_ref, out_ref):
    ty = ty_ref[...]
    tx = tx_ref[...]
    th = th_ref[...]
    tw = tw_ref[...]
    ay1 = ay1_ref[...]
    ax1 = ax1_ref[...]
    ay2 = ay2_ref[...]
    ax2 = ax2_ref[...]
    clsf = cls_ref[...]

    yca = (ay1 + ay2) * 0.5
    xca = (ax1 + ax2) * 0.5
    ha = ay2 - ay1
    wa = ax2 - ax1
    h = jnp.exp(th) * ha
    w = jnp.exp(tw) * wa
    yc = ty * ha + yca
    xc = tx * wa + xca
    by1 = yc - h * 0.5
    bx1 = xc - w * 0.5
    by2 = yc + h * 0.5
    bx2 = xc + w * 0.5

    lane = jax.lax.broadcasted_iota(jnp.int32, (NB, NP_PAD), 1)
    valid = lane < N_TOP
    scores0 = jnp.where(valid, jax.nn.sigmoid(logit_ref[...]), -1e30)

    off = clsf * 1e4
    oy1 = by1 + off
    ox1 = bx1 + off
    oy2 = by2 + off
    ox2 = bx2 + off
    area_o = (oy2 - oy1) * (ox2 - ox1)
    scale = scale_ref[...]

    def step(i, scores):
        m = jnp.max(scores, axis=1, keepdims=True)
        cand = jnp.where(scores == m, lane, NP_PAD)
        bidx = jnp.min(cand, axis=1, keepdims=True)
        sel = lane == bidx

        def pick(a):
            return jnp.sum(jnp.where(sel, a, 0.0), axis=1, keepdims=True)

        b_oy1 = pick(oy1)
        b_ox1 = pick(ox1)
        b_oy2 = pick(oy2)
        b_ox2 = pick(ox2)
        ya = jnp.maximum(b_oy1, oy1)
        xa = jnp.maximum(b_ox1, ox1)
        yb = jnp.minimum(b_oy2, oy2)
        xb = jnp.minimum(b_ox2, ox2)
        inter = jnp.maximum(yb - ya, 0.0) * jnp.maximum(xb - xa, 0.0)
        a1 = (b_oy2 - b_oy1) * (b_ox2 - b_ox1)
        iou = inter / (a1 + area_o - inter + 1e-8)

        b_score = pick(scores)
        det = jnp.concatenate([
            pick(by1) * scale, pick(bx1) * scale,
            pick(by2) * scale, pick(bx2) * scale,
            b_score, pick(clsf) + 1.0,
        ], axis=1)
        out_ref[:, pl.ds(i, 1), :] = det[:, None, :]

        scores = jnp.where(iou > IOU_T, -1.0, scores)
        return jnp.where(sel, -1.0, scores)

    jax.lax.fori_loop(0, N_DET, step, scores0)


@jax.jit
def kernel(cls_p3, cls_p4, cls_p5, cls_p6, cls_p7,
           box_p3, box_p4, box_p5, box_p6, box_p7, image_scales):
    cls_list = [cls_p3, cls_p4, cls_p5, cls_p6, cls_p7]
    box_list = [box_p3, box_p4, box_p5, box_p6, box_p7]
    b = cls_p3.shape[0]
    cls_flat = jnp.concatenate(
        [jnp.transpose(c, (0, 2, 3, 1)).reshape(b, -1) for c in cls_list], axis=1)
    top_v, topi = jax.lax.top_k(cls_flat, N_TOP)
    indices = topi // N_CLASSES
    classes = topi % N_CLASSES

    pad_fill = jnp.arange(G_PAD - G_TOT, dtype=jnp.int32)
    aidx = jnp.concatenate([indices.reshape(-1), pad_fill])
    bidx = jnp.concatenate(
        [(indices + jnp.arange(b, dtype=jnp.int32)[:, None] * N_ANCH).reshape(-1),
         pad_fill])
    btabs = []
    for j in range(4):
        btabs.append(jnp.concatenate(
            [jnp.transpose(o[:, j::4], (0, 2, 3, 1)).reshape(b, -1)
             for o in box_list], axis=1).reshape(-1))
    atabs = [jnp.asarray(np.ascontiguousarray(_ANCH[:, j])) for j in range(4)]
    gat = _sc_gather_fn()(*btabs, *atabs,
                     bidx.reshape(_NW, G_NCH, G_CHUNK),
                     aidx.reshape(_NW, G_NCH, G_CHUNK))

    def unflat(x):
        return x[:G_TOT].reshape(b, N_TOP)

    args = [
        unflat(gat[0]), unflat(gat[1]), unflat(gat[2]), unflat(gat[3]),
        unflat(gat[4]), unflat(gat[5]), unflat(gat[6]), unflat(gat[7]),
        top_v, classes.astype(jnp.float32),
        image_scales[:, None],
    ]
    dets = pl.pallas_call(
        _nms_body,
        out_shape=jax.ShapeDtypeStruct((NB, N_DET, 6), jnp.float32),
    )(*args)
    return dets
